# baseline (speedup 1.0000x reference)
import functools

import jax
import jax.numpy as jnp
from jax import lax
from jax.experimental import pallas as pl
from jax.experimental.pallas import tpu as pltpu
from jax.experimental.pallas import tpu_sc as plsc

N_NODES = 10000
D = 128
N_EDGES = 320000

NC = 2
NS = 16
NW = NC * NS
CHUNK = 128
NCH0 = 66
NCH1 = 92
EPW0 = NCH0 * CHUNK
EPW1 = NCH1 * CHUNK
E_PAD = NS * (EPW0 + EPW1)
NROWS = 10240
RPT = NROWS // NS
TRASH = N_NODES
BLK = 1024


def _sc_degree_body(dst_hbm, deg_out_hbm, degacc, obuf, zbuf, fbuf, idxv):
    cid = lax.axis_index("c")
    sid = lax.axis_index("s")
    base0 = cid * NS * EPW0 + sid * jnp.where(cid == 0, EPW0, EPW1)
    nch = jnp.where(cid == 0, NCH0, NCH1)

    def fill(r, carry):
        obuf[r, :] = jnp.ones((16,), jnp.float32)
        zbuf[r, :] = jnp.zeros((16,), jnp.float32)
        return carry

    lax.fori_loop(0, CHUNK, fill, 0)
    for c in range(RPT // CHUNK):
        pltpu.sync_copy(zbuf, degacc.at[pl.ds(sid * RPT + c * CHUNK, CHUNK)])
    plsc.subcore_barrier()

    def body(c, carry):
        base = base0 + c * CHUNK
        pltpu.sync_copy(dst_hbm.at[pl.ds(base, CHUNK)], idxv)
        pltpu.sync_copy(obuf, degacc.at[idxv], add=True)
        return carry

    lax.fori_loop(0, nch, body, 0)
    plsc.subcore_barrier()

    for c in range(RPT // CHUNK):
        r0 = sid * RPT + c * CHUNK
        pltpu.sync_copy(degacc.at[pl.ds(r0, CHUNK)], zbuf)

        def repack(j, carry):
            for k in range(8):
                fbuf[j, pl.ds(k * 16, 16)] = zbuf[8 * j + k, :]
            return carry

        lax.fori_loop(0, 16, repack, 0)
        orow = pl.multiple_of(sid * (RPT // 8) + c * 16, 8)
        pltpu.sync_copy(fbuf, deg_out_hbm.at[cid, pl.ds(orow, 16)])


def _sc_segsum_body(xw_hbm, src_hbm, dst_hbm, zeros128_hbm, part_hbm,
                    acc, rows, sidx, didx, sem):
    cid = lax.axis_index("c")
    sid = lax.axis_index("s")
    base0 = cid * NS * EPW0 + sid * jnp.where(cid == 0, EPW0, EPW1)
    nch = jnp.where(cid == 0, NCH0, NCH1)

    pltpu.sync_copy(zeros128_hbm, rows)
    for c in range(RPT // CHUNK):
        pltpu.sync_copy(rows, acc.at[pl.ds(sid * RPT + c * CHUNK, CHUNK)])
    plsc.subcore_barrier()

    def body(c, carry):
        base = base0 + c * CHUNK
        pltpu.sync_copy(src_hbm.at[pl.ds(base, CHUNK)], sidx)
        pltpu.sync_copy(dst_hbm.at[pl.ds(base, CHUNK)], didx)
        pltpu.async_copy(xw_hbm.at[sidx], rows, sem).wait()
        pltpu.sync_copy(rows, acc.at[didx], add=True)
        return carry

    lax.fori_loop(0, nch, body, 0)
    plsc.subcore_barrier()

    for c in range(RPT // CHUNK):
        r0 = sid * RPT + c * CHUNK
        pltpu.sync_copy(acc.at[pl.ds(r0, CHUNK)], rows)
        pltpu.sync_copy(rows, part_hbm.at[cid, pl.ds(r0, CHUNK)])


_SC_MESH = plsc.VectorSubcoreMesh(
    core_axis_name="c", subcore_axis_name="s", num_cores=NC, num_subcores=NS)

_sc_degree = pl.kernel(
    _sc_degree_body,
    out_type=jax.ShapeDtypeStruct((NC, NROWS // 8, D), jnp.float32),
    mesh=_SC_MESH,
    scratch_types=[
        pltpu.VMEM_SHARED((NROWS, 16), jnp.float32),
        pltpu.VMEM((CHUNK, 16), jnp.float32),
        pltpu.VMEM((CHUNK, 16), jnp.float32),
        pltpu.VMEM((16, D), jnp.float32),
        pltpu.VMEM((CHUNK,), jnp.int32),
    ],
)

_sc_segsum = pl.kernel(
    _sc_segsum_body,
    out_type=jax.ShapeDtypeStruct((NC, NROWS, D), jnp.float32),
    mesh=_SC_MESH,
    scratch_types=[
        pltpu.VMEM_SHARED((NROWS, D), jnp.float32),
        pltpu.VMEM((CHUNK, D), jnp.float32),
        pltpu.VMEM((CHUNK,), jnp.int32),
        pltpu.VMEM((CHUNK,), jnp.int32),
        pltpu.SemaphoreType.DMA,
    ],
)


def _dinv_of(deg_ref):
    deg = deg_ref[0, :, 0] + deg_ref[1, :, 0] + 1.0
    return lax.rsqrt(deg)


def _tc_prep_body(x_ref, w_ref, deg_ref, o_ref):
    dinv = _dinv_of(deg_ref)
    xw = jnp.dot(x_ref[...], w_ref[...], preferred_element_type=jnp.float32)
    o_ref[...] = xw * dinv[:, None]


def _tc_mid_body(part_ref, xwp_ref, deg_ref, b_ref, w_ref, o_ref):
    dinv = _dinv_of(deg_ref)
    s = part_ref[0] + part_ref[1] + xwp_ref[...]
    h = jnp.maximum(s * dinv[:, None] + b_ref[...], 0.0)
    o_ref[...] = jnp.dot(h, w_ref[...],
                         preferred_element_type=jnp.float32) * dinv[:, None]


def _tc_final_body(part_ref, xwp_ref, deg_ref, b_ref, o_ref):
    dinv = _dinv_of(deg_ref)
    s = part_ref[0] + part_ref[1] + xwp_ref[...]
    o_ref[...] = jnp.maximum(s * dinv[:, None] + b_ref[...], 0.0)


_GRID = (NROWS // BLK,)
_row_spec = pl.BlockSpec((BLK, D), lambda i: (i, 0))
_part_spec = pl.BlockSpec((NC, BLK, D), lambda i: (0, i, 0))
_deg_spec = pl.BlockSpec((NC, BLK, 16), lambda i: (0, i, 0))
_w_spec = pl.BlockSpec((D, D), lambda i: (0, 0))
_b_spec = pl.BlockSpec((1, D), lambda i: (0, 0))

_tc_prep = pl.pallas_call(
    _tc_prep_body, grid=_GRID,
    in_specs=[_row_spec, _w_spec, _deg_spec],
    out_specs=_row_spec,
    out_shape=jax.ShapeDtypeStruct((NROWS, D), jnp.float32),
)

_tc_mid = pl.pallas_call(
    _tc_mid_body, grid=_GRID,
    in_specs=[_part_spec, _row_spec, _deg_spec, _b_spec, _w_spec],
    out_specs=_row_spec,
    out_shape=jax.ShapeDtypeStruct((NROWS, D), jnp.float32),
)

_tc_final = pl.pallas_call(
    _tc_final_body, grid=_GRID,
    in_specs=[_part_spec, _row_spec, _deg_spec, _b_spec],
    out_specs=_row_spec,
    out_shape=jax.ShapeDtypeStruct((NROWS, D), jnp.float32),
)


def kernel(x, edge_index, W1, b1, W2, b2):
    src = edge_index[0].astype(jnp.int32)
    dst = edge_index[1].astype(jnp.int32)
    pad = E_PAD - N_EDGES
    src_p = jnp.concatenate([src, jnp.zeros((pad,), jnp.int32)])
    dst_p = jnp.concatenate([dst, jnp.full((pad,), TRASH, jnp.int32)])
    x_p = jnp.pad(x, ((0, NROWS - N_NODES), (0, 0)))

    zeros128 = jnp.zeros((CHUNK, D), jnp.float32)

    degcnt = _sc_degree(dst_p).reshape(NC, NROWS, 16)
    xw1p = _tc_prep(x_p, W1, degcnt)
    part1 = _sc_segsum(xw1p, src_p, dst_p, zeros128)
    xw2p = _tc_mid(part1, xw1p, degcnt, b1.reshape(1, D), W2)
    part2 = _sc_segsum(xw2p, src_p, dst_p, zeros128)
    out = _tc_final(part2, xw2p, degcnt, b2.reshape(1, D))
    return out[:N_NODES]


# speedup vs baseline: 1.1490x; 1.1490x over previous
import functools

import jax
import jax.numpy as jnp
from jax import lax
from jax.experimental import pallas as pl
from jax.experimental.pallas import tpu as pltpu
from jax.experimental.pallas import tpu_sc as plsc

N_NODES = 10000
D = 128
N_EDGES = 320000

NC = 2
NS = 16
NW = NC * NS
CHUNK = 128
NCH0 = 92
NCH1 = 66
EPW0 = NCH0 * CHUNK
EPW1 = NCH1 * CHUNK
E_PAD = NS * (EPW0 + EPW1)
NROWS = 10240
RPT = NROWS // NS
TRASH = N_NODES
BLK = 1024


def _sc_degree_body(dst_hbm, deg_out_hbm, degacc, obuf, zbuf, fbuf, idxv):
    cid = lax.axis_index("c")
    sid = lax.axis_index("s")
    base0 = cid * NS * EPW0 + sid * jnp.where(cid == 0, EPW0, EPW1)
    nch = jnp.where(cid == 0, NCH0, NCH1)

    def fill(r, carry):
        obuf[r, :] = jnp.ones((16,), jnp.float32)
        zbuf[r, :] = jnp.zeros((16,), jnp.float32)
        return carry

    lax.fori_loop(0, CHUNK, fill, 0)
    for c in range(RPT // CHUNK):
        pltpu.sync_copy(zbuf, degacc.at[pl.ds(sid * RPT + c * CHUNK, CHUNK)])
    plsc.subcore_barrier()

    def body(c, carry):
        base = base0 + c * CHUNK
        pltpu.sync_copy(dst_hbm.at[pl.ds(base, CHUNK)], idxv)
        pltpu.sync_copy(obuf, degacc.at[idxv], add=True)
        return carry

    lax.fori_loop(0, nch, body, 0)
    plsc.subcore_barrier()

    for c in range(RPT // CHUNK):
        r0 = sid * RPT + c * CHUNK
        pltpu.sync_copy(degacc.at[pl.ds(r0, CHUNK)], zbuf)

        def repack(j, carry):
            for k in range(8):
                fbuf[j, pl.ds(k * 16, 16)] = zbuf[8 * j + k, :]
            return carry

        lax.fori_loop(0, 16, repack, 0)
        orow = pl.multiple_of(sid * (RPT // 8) + c * 16, 8)
        pltpu.sync_copy(fbuf, deg_out_hbm.at[cid, pl.ds(orow, 16)])


def _sc_segsum_body(xw_hbm, src_hbm, dst_hbm, zeros128_hbm, part_hbm,
                    acc, rows, sidx, didx, sem):
    cid = lax.axis_index("c")
    sid = lax.axis_index("s")
    base0 = cid * NS * EPW0 + sid * jnp.where(cid == 0, EPW0, EPW1)
    nch = jnp.where(cid == 0, NCH0, NCH1)

    pltpu.sync_copy(zeros128_hbm, rows)
    for c in range(RPT // CHUNK):
        pltpu.sync_copy(rows, acc.at[pl.ds(sid * RPT + c * CHUNK, CHUNK)])
    plsc.subcore_barrier()

    def body(c, carry):
        base = base0 + c * CHUNK
        pltpu.sync_copy(src_hbm.at[pl.ds(base, CHUNK)], sidx)
        pltpu.sync_copy(dst_hbm.at[pl.ds(base, CHUNK)], didx)
        pltpu.async_copy(xw_hbm.at[sidx], rows, sem).wait()
        pltpu.sync_copy(rows, acc.at[didx], add=True)
        return carry

    lax.fori_loop(0, nch, body, 0)
    plsc.subcore_barrier()

    for c in range(RPT // CHUNK):
        r0 = sid * RPT + c * CHUNK
        pltpu.sync_copy(acc.at[pl.ds(r0, CHUNK)], rows)
        pltpu.sync_copy(rows, part_hbm.at[cid, pl.ds(r0, CHUNK)])


_SC_MESH = plsc.VectorSubcoreMesh(
    core_axis_name="c", subcore_axis_name="s", num_cores=NC, num_subcores=NS)

_sc_degree = pl.kernel(
    _sc_degree_body,
    out_type=jax.ShapeDtypeStruct((NC, NROWS // 8, D), jnp.float32),
    mesh=_SC_MESH,
    scratch_types=[
        pltpu.VMEM_SHARED((NROWS, 16), jnp.float32),
        pltpu.VMEM((CHUNK, 16), jnp.float32),
        pltpu.VMEM((CHUNK, 16), jnp.float32),
        pltpu.VMEM((16, D), jnp.float32),
        pltpu.VMEM((CHUNK,), jnp.int32),
    ],
)

_sc_segsum = pl.kernel(
    _sc_segsum_body,
    out_type=jax.ShapeDtypeStruct((NC, NROWS, D), jnp.float32),
    mesh=_SC_MESH,
    scratch_types=[
        pltpu.VMEM_SHARED((NROWS, D), jnp.float32),
        pltpu.VMEM((CHUNK, D), jnp.float32),
        pltpu.VMEM((CHUNK,), jnp.int32),
        pltpu.VMEM((CHUNK,), jnp.int32),
        pltpu.SemaphoreType.DMA,
    ],
)


def _dinv_of(deg_ref):
    deg = deg_ref[0, :, 0] + deg_ref[1, :, 0] + 1.0
    return lax.rsqrt(deg)


def _tc_prep_body(x_ref, w_ref, deg_ref, o_ref):
    dinv = _dinv_of(deg_ref)
    xw = jnp.dot(x_ref[...], w_ref[...], preferred_element_type=jnp.float32)
    o_ref[...] = xw * dinv[:, None]


def _tc_mid_body(part_ref, xwp_ref, deg_ref, b_ref, w_ref, o_ref):
    dinv = _dinv_of(deg_ref)
    s = part_ref[0] + part_ref[1] + xwp_ref[...]
    h = jnp.maximum(s * dinv[:, None] + b_ref[...], 0.0)
    o_ref[...] = jnp.dot(h, w_ref[...],
                         preferred_element_type=jnp.float32) * dinv[:, None]


def _tc_final_body(part_ref, xwp_ref, deg_ref, b_ref, o_ref):
    dinv = _dinv_of(deg_ref)
    s = part_ref[0] + part_ref[1] + xwp_ref[...]
    o_ref[...] = jnp.maximum(s * dinv[:, None] + b_ref[...], 0.0)


_GRID = (NROWS // BLK,)
_row_spec = pl.BlockSpec((BLK, D), lambda i: (i, 0))
_part_spec = pl.BlockSpec((NC, BLK, D), lambda i: (0, i, 0))
_deg_spec = pl.BlockSpec((NC, BLK, 16), lambda i: (0, i, 0))
_w_spec = pl.BlockSpec((D, D), lambda i: (0, 0))
_b_spec = pl.BlockSpec((1, D), lambda i: (0, 0))

_tc_prep = pl.pallas_call(
    _tc_prep_body, grid=_GRID,
    in_specs=[_row_spec, _w_spec, _deg_spec],
    out_specs=_row_spec,
    out_shape=jax.ShapeDtypeStruct((NROWS, D), jnp.float32),
)

_tc_mid = pl.pallas_call(
    _tc_mid_body, grid=_GRID,
    in_specs=[_part_spec, _row_spec, _deg_spec, _b_spec, _w_spec],
    out_specs=_row_spec,
    out_shape=jax.ShapeDtypeStruct((NROWS, D), jnp.float32),
)

_tc_final = pl.pallas_call(
    _tc_final_body, grid=_GRID,
    in_specs=[_part_spec, _row_spec, _deg_spec, _b_spec],
    out_specs=_row_spec,
    out_shape=jax.ShapeDtypeStruct((NROWS, D), jnp.float32),
)


def kernel(x, edge_index, W1, b1, W2, b2):
    src = edge_index[0].astype(jnp.int32)
    dst = edge_index[1].astype(jnp.int32)
    pad = E_PAD - N_EDGES
    src_p = jnp.concatenate([src, jnp.zeros((pad,), jnp.int32)])
    dst_p = jnp.concatenate([dst, jnp.full((pad,), TRASH, jnp.int32)])
    x_p = jnp.pad(x, ((0, NROWS - N_NODES), (0, 0)))

    zeros128 = jnp.zeros((CHUNK, D), jnp.float32)

    degcnt = _sc_degree(dst_p).reshape(NC, NROWS, 16)
    xw1p = _tc_prep(x_p, W1, degcnt)
    part1 = _sc_segsum(xw1p, src_p, dst_p, zeros128)
    xw2p = _tc_mid(part1, xw1p, degcnt, b1.reshape(1, D), W2)
    part2 = _sc_segsum(xw2p, src_p, dst_p, zeros128)
    out = _tc_final(part2, xw2p, degcnt, b2.reshape(1, D))
    return out[:N_NODES]


# speedup vs baseline: 1.2454x; 1.0839x over previous
import functools

import jax
import jax.numpy as jnp
from jax import lax
from jax.experimental import pallas as pl
from jax.experimental.pallas import tpu as pltpu
from jax.experimental.pallas import tpu_sc as plsc

N_NODES = 10000
D = 128
N_EDGES = 320000

NC = 2
NS = 16
NW = NC * NS
CHUNK = 128
NCH0 = 92
NCH1 = 66
EPW0 = NCH0 * CHUNK
EPW1 = NCH1 * CHUNK
E_PAD = NS * (EPW0 + EPW1)
EPW_U = E_PAD // NW
NCH_U = EPW_U // CHUNK
EXTRA = 2 * CHUNK
NROWS = 10240
RPT = NROWS // NS
TRASH = N_NODES
BLK = 1024


def _sc_degree_body(dst_hbm, deg_out_hbm, degacc, obuf, zbuf, fbuf, idxv):
    cid = lax.axis_index("c")
    sid = lax.axis_index("s")
    w = cid * NS + sid

    def fill(r, carry):
        obuf[r, :] = jnp.ones((16,), jnp.float32)
        zbuf[r, :] = jnp.zeros((16,), jnp.float32)
        return carry

    lax.fori_loop(0, CHUNK, fill, 0)
    for c in range(RPT // CHUNK):
        pltpu.sync_copy(zbuf, degacc.at[pl.ds(sid * RPT + c * CHUNK, CHUNK)])
    plsc.subcore_barrier()

    def body(c, carry):
        base = w * EPW_U + c * CHUNK
        pltpu.sync_copy(dst_hbm.at[pl.ds(base, CHUNK)], idxv)
        pltpu.sync_copy(obuf, degacc.at[idxv], add=True)
        return carry

    lax.fori_loop(0, NCH_U, body, 0)
    plsc.subcore_barrier()

    for c in range(RPT // CHUNK):
        r0 = sid * RPT + c * CHUNK
        pltpu.sync_copy(degacc.at[pl.ds(r0, CHUNK)], zbuf)

        def repack(j, carry):
            for k in range(8):
                fbuf[j, pl.ds(k * 16, 16)] = zbuf[8 * j + k, :]
            return carry

        lax.fori_loop(0, 16, repack, 0)
        orow = pl.multiple_of(sid * (RPT // 8) + c * 16, 8)
        pltpu.sync_copy(fbuf, deg_out_hbm.at[cid, pl.ds(orow, 16)])


def _sc_segsum_body(xw_hbm, src_hbm, dst_hbm, zeros128_hbm, part_hbm,
                    acc, rows, sidx0, didx0, sidx1, didx1,
                    sem, isem0, isem1):
    cid = lax.axis_index("c")
    sid = lax.axis_index("s")
    base0 = cid * NS * EPW0 + sid * jnp.where(cid == 0, EPW0, EPW1)
    nch = jnp.where(cid == 0, NCH0, NCH1)

    pltpu.sync_copy(zeros128_hbm, rows)
    for c in range(RPT // CHUNK):
        pltpu.sync_copy(rows, acc.at[pl.ds(sid * RPT + c * CHUNK, CHUNK)])
    plsc.subcore_barrier()

    pltpu.async_copy(src_hbm.at[pl.ds(base0, CHUNK)], sidx0, isem0).wait()
    pltpu.async_copy(dst_hbm.at[pl.ds(base0, CHUNK)], didx0, isem0).wait()
    pltpu.async_copy(src_hbm.at[pl.ds(base0 + CHUNK, CHUNK)], sidx1, isem1)
    pltpu.async_copy(dst_hbm.at[pl.ds(base0 + CHUNK, CHUNK)], didx1, isem1)

    def body(i, carry):
        a = 2 * i
        pltpu.async_copy(xw_hbm.at[sidx0], rows, sem).wait()
        pltpu.sync_copy(rows, acc.at[didx0], add=True)
        pltpu.async_copy(src_hbm.at[pl.ds(base0 + (a + 2) * CHUNK, CHUNK)],
                         sidx0, isem0)
        pltpu.async_copy(dst_hbm.at[pl.ds(base0 + (a + 2) * CHUNK, CHUNK)],
                         didx0, isem0)
        pltpu.make_async_copy(src_hbm.at[pl.ds(0, CHUNK)], sidx1, isem1).wait()
        pltpu.make_async_copy(dst_hbm.at[pl.ds(0, CHUNK)], didx1, isem1).wait()
        pltpu.async_copy(xw_hbm.at[sidx1], rows, sem).wait()
        pltpu.sync_copy(rows, acc.at[didx1], add=True)
        pltpu.async_copy(src_hbm.at[pl.ds(base0 + (a + 3) * CHUNK, CHUNK)],
                         sidx1, isem1)
        pltpu.async_copy(dst_hbm.at[pl.ds(base0 + (a + 3) * CHUNK, CHUNK)],
                         didx1, isem1)
        pltpu.make_async_copy(src_hbm.at[pl.ds(0, CHUNK)], sidx0, isem0).wait()
        pltpu.make_async_copy(dst_hbm.at[pl.ds(0, CHUNK)], didx0, isem0).wait()
        return carry

    lax.fori_loop(0, nch // 2, body, 0)
    pltpu.make_async_copy(src_hbm.at[pl.ds(0, CHUNK)], sidx1, isem1).wait()
    pltpu.make_async_copy(dst_hbm.at[pl.ds(0, CHUNK)], didx1, isem1).wait()
    plsc.subcore_barrier()

    for c in range(RPT // CHUNK):
        r0 = sid * RPT + c * CHUNK
        pltpu.sync_copy(acc.at[pl.ds(r0, CHUNK)], rows)
        pltpu.sync_copy(rows, part_hbm.at[cid, pl.ds(r0, CHUNK)])


_SC_MESH = plsc.VectorSubcoreMesh(
    core_axis_name="c", subcore_axis_name="s", num_cores=NC, num_subcores=NS)

_sc_degree = pl.kernel(
    _sc_degree_body,
    out_type=jax.ShapeDtypeStruct((NC, NROWS // 8, D), jnp.float32),
    mesh=_SC_MESH,
    scratch_types=[
        pltpu.VMEM_SHARED((NROWS, 16), jnp.float32),
        pltpu.VMEM((CHUNK, 16), jnp.float32),
        pltpu.VMEM((CHUNK, 16), jnp.float32),
        pltpu.VMEM((16, D), jnp.float32),
        pltpu.VMEM((CHUNK,), jnp.int32),
    ],
)

_sc_segsum = pl.kernel(
    _sc_segsum_body,
    out_type=jax.ShapeDtypeStruct((NC, NROWS, D), jnp.float32),
    mesh=_SC_MESH,
    scratch_types=[
        pltpu.VMEM_SHARED((NROWS, D), jnp.float32),
        pltpu.VMEM((CHUNK, D), jnp.float32),
        pltpu.VMEM((CHUNK,), jnp.int32),
        pltpu.VMEM((CHUNK,), jnp.int32),
        pltpu.VMEM((CHUNK,), jnp.int32),
        pltpu.VMEM((CHUNK,), jnp.int32),
        pltpu.SemaphoreType.DMA,
        pltpu.SemaphoreType.DMA,
        pltpu.SemaphoreType.DMA,
    ],
)


def _dinv_of(deg_ref):
    deg = deg_ref[0, :, 0] + deg_ref[1, :, 0] + 1.0
    return lax.rsqrt(deg)


def _tc_prep_body(x_ref, w_ref, deg_ref, o_ref):
    dinv = _dinv_of(deg_ref)
    xw = jnp.dot(x_ref[...], w_ref[...], preferred_element_type=jnp.float32)
    o_ref[...] = xw * dinv[:, None]


def _tc_mid_body(part_ref, xwp_ref, deg_ref, b_ref, w_ref, o_ref):
    dinv = _dinv_of(deg_ref)
    s = part_ref[0] + part_ref[1] + xwp_ref[...]
    h = jnp.maximum(s * dinv[:, None] + b_ref[...], 0.0)
    o_ref[...] = jnp.dot(h, w_ref[...],
                         preferred_element_type=jnp.float32) * dinv[:, None]


def _tc_final_body(part_ref, xwp_ref, deg_ref, b_ref, o_ref):
    dinv = _dinv_of(deg_ref)
    s = part_ref[0] + part_ref[1] + xwp_ref[...]
    o_ref[...] = jnp.maximum(s * dinv[:, None] + b_ref[...], 0.0)


_GRID = (NROWS // BLK,)
_row_spec = pl.BlockSpec((BLK, D), lambda i: (i, 0))
_part_spec = pl.BlockSpec((NC, BLK, D), lambda i: (0, i, 0))
_deg_spec = pl.BlockSpec((NC, BLK, 16), lambda i: (0, i, 0))
_w_spec = pl.BlockSpec((D, D), lambda i: (0, 0))
_b_spec = pl.BlockSpec((1, D), lambda i: (0, 0))

_tc_prep = pl.pallas_call(
    _tc_prep_body, grid=_GRID,
    in_specs=[_row_spec, _w_spec, _deg_spec],
    out_specs=_row_spec,
    out_shape=jax.ShapeDtypeStruct((NROWS, D), jnp.float32),
)

_tc_mid = pl.pallas_call(
    _tc_mid_body, grid=_GRID,
    in_specs=[_part_spec, _row_spec, _deg_spec, _b_spec, _w_spec],
    out_specs=_row_spec,
    out_shape=jax.ShapeDtypeStruct((NROWS, D), jnp.float32),
)

_tc_final = pl.pallas_call(
    _tc_final_body, grid=_GRID,
    in_specs=[_part_spec, _row_spec, _deg_spec, _b_spec],
    out_specs=_row_spec,
    out_shape=jax.ShapeDtypeStruct((NROWS, D), jnp.float32),
)


def kernel(x, edge_index, W1, b1, W2, b2):
    src = edge_index[0].astype(jnp.int32)
    dst = edge_index[1].astype(jnp.int32)
    pad = E_PAD + EXTRA - N_EDGES
    src_p = jnp.concatenate([src, jnp.zeros((pad,), jnp.int32)])
    dst_p = jnp.concatenate([dst, jnp.full((pad,), TRASH, jnp.int32)])
    x_p = jnp.pad(x, ((0, NROWS - N_NODES), (0, 0)))

    zeros128 = jnp.zeros((CHUNK, D), jnp.float32)

    degcnt = _sc_degree(dst_p).reshape(NC, NROWS, 16)
    xw1p = _tc_prep(x_p, W1, degcnt)
    part1 = _sc_segsum(xw1p, src_p, dst_p, zeros128)
    xw2p = _tc_mid(part1, xw1p, degcnt, b1.reshape(1, D), W2)
    part2 = _sc_segsum(xw2p, src_p, dst_p, zeros128)
    out = _tc_final(part2, xw2p, degcnt, b2.reshape(1, D))
    return out[:N_NODES]
